# baseline (speedup 1.0000x reference)
import functools

import jax
import jax.numpy as jnp
from jax.experimental import pallas as pl
from jax.experimental.pallas import tpu as pltpu

LN_EPS = 1e-5

_F32 = jnp.float32
_BF16 = jnp.bfloat16

_G = 512
_TT = 256
_VT = 1024
_NBUF = 8


def _gather_ln_body(idx_ref, emb_ref, g_ref, b_ref, x32_ref, xbf_ref,
                    xg, sem, *, g: int):
    t0 = pl.program_id(0) * g
    for r in range(g):
        pltpu.make_async_copy(
            emb_ref.at[idx_ref[t0 + r]], xg.at[r], sem
        ).start()
    for r in range(g):
        pltpu.make_async_copy(
            emb_ref.at[idx_ref[t0 + r]], xg.at[r], sem
        ).wait()
    rows = xg[...]
    mu = jnp.mean(rows, axis=1, keepdims=True)
    d = rows - mu
    var = jnp.mean(d * d, axis=1, keepdims=True)
    y = d * jax.lax.rsqrt(var + LN_EPS) * g_ref[...] + b_ref[...]
    x32_ref[...] = y
    xbf_ref[...] = y.astype(_BF16)


def _gather_ln(idx_flat, emb_w, ln_g2, ln_b2):
    n, e = idx_flat.shape[0], emb_w.shape[1]
    grid = (n // _G,)
    in_specs = [
        pl.BlockSpec(memory_space=pl.ANY),
        pl.BlockSpec((1, e), lambda i, idx_ref: (0, 0)),
        pl.BlockSpec((1, e), lambda i, idx_ref: (0, 0)),
    ]
    out_specs = [
        pl.BlockSpec((_G, e), lambda i, idx_ref: (i, 0)),
        pl.BlockSpec((_G, e), lambda i, idx_ref: (i, 0)),
    ]
    return pl.pallas_call(
        functools.partial(_gather_ln_body, g=_G),
        grid_spec=pltpu.PrefetchScalarGridSpec(
            num_scalar_prefetch=1,
            grid=grid,
            in_specs=in_specs,
            out_specs=out_specs,
            scratch_shapes=[
                pltpu.VMEM((_G, e), _F32),
                pltpu.SemaphoreType.DMA,
            ],
        ),
        out_shape=[
            jax.ShapeDtypeStruct((n, e), _F32),
            jax.ShapeDtypeStruct((n, e), _BF16),
        ],
        compiler_params=pltpu.CompilerParams(
            dimension_semantics=("arbitrary",),
        ),
        name="gather_ln",
    )(idx_flat, emb_w, ln_g2, ln_b2)


_CDIMS_11 = (((1,), (1,)), ((), ()))
_CDIMS_10 = (((1,), (0,)), ((), ()))


def _qkc_body(x_ref, wq_ref, wk_ref, c_ref, k_s, *, t: int, qk: int):
    k_s[...] = jax.lax.dot_general(
        x_ref[0], wk_ref[...], _CDIMS_11, preferred_element_type=_F32
    ).astype(_BF16)
    inv_qk = _F32(1.0 / qk)
    for i in range(t // _TT):
        sl = slice(i * _TT, (i + 1) * _TT)
        ki = (i + 1) * _TT
        qi = jax.lax.dot_general(
            x_ref[0, sl], wq_ref[...], _CDIMS_11, preferred_element_type=_F32
        ).astype(_BF16)
        ci = jax.lax.dot_general(
            qi, k_s[:ki], _CDIMS_11, preferred_element_type=_F32
        ) * inv_qk
        row = i * _TT + jax.lax.broadcasted_iota(jnp.int32, (_TT, ki), 0)
        col = jax.lax.broadcasted_iota(jnp.int32, (_TT, ki), 1)
        ci = jnp.where(row >= col, ci, _F32(0.0))
        c_ref[0, sl, :ki] = ci.astype(_BF16)
        if ki < t:
            c_ref[0, sl, ki:] = jnp.zeros((_TT, t - ki), _BF16)


def _qkc(xbf, wq, wk):
    b, t, e = xbf.shape
    qk = wq.shape[0]
    return pl.pallas_call(
        functools.partial(_qkc_body, t=t, qk=qk),
        grid=(b,),
        in_specs=[
            pl.BlockSpec((1, t, e), lambda i: (i, 0, 0)),
            pl.BlockSpec((qk, e), lambda i: (0, 0)),
            pl.BlockSpec((qk, e), lambda i: (0, 0)),
        ],
        out_specs=pl.BlockSpec((1, t, t), lambda i: (i, 0, 0)),
        out_shape=jax.ShapeDtypeStruct((b, t, t), _BF16),
        scratch_shapes=[pltpu.VMEM((t, qk), _BF16)],
        compiler_params=pltpu.CompilerParams(
            dimension_semantics=("arbitrary",),
        ),
        name="qkc",
    )(xbf, wq, wk)


def _logits_body(idx_ref, x_ref, c_ref, hw_ref, o_ref,
                 oh_s, ob, ob_t, osem, *, b: int, t: int, v: int, nv: int):
    step = pl.program_id(0)
    v0 = step * _VT
    tail = v - (nv - 1) * _VT
    is_tail = step == nv - 1
    not_tail = step < nv - 1
    hw = hw_ref[...].astype(_BF16)
    col_ids = v0 + jax.lax.broadcasted_iota(jnp.int32, (_VT, t), 0)
    nt = t // _TT

    def _o_copy(src, slot, bi, t0, rows):
        return pltpu.make_async_copy(
            src,
            o_ref.at[pl.ds(v0, rows), bi, pl.ds(t0, _TT)],
            osem.at[slot],
        )

    def _o_both(j, meth):
        slot, bi, t0 = j % _NBUF, j // nt, (j % nt) * _TT

        @pl.when(not_tail)
        def _():
            meth(_o_copy(ob.at[slot], slot, bi, t0, _VT))

        @pl.when(is_tail)
        def _():
            meth(_o_copy(ob_t.at[slot], slot, bi, t0, tail))

    for bi in range(b):
        ids = idx_ref[bi]
        oh_s[...] = jnp.where(ids == col_ids, _F32(1.0), _F32(0.0)).astype(_BF16)
        for ti in range(nt):
            j = bi * nt + ti
            sl = slice(ti * _TT, (ti + 1) * _TT)
            ki = (ti + 1) * _TT
            head_t = jax.lax.dot_general(
                hw, x_ref[bi, sl], _CDIMS_11, preferred_element_type=_F32
            )
            cpy_t = jax.lax.dot_general(
                oh_s[:, :ki], c_ref[bi, sl, :ki], _CDIMS_11,
                preferred_element_type=_F32,
            )
            if j >= _NBUF:
                _o_both(j - _NBUF, lambda d: d.wait())
            res = head_t + cpy_t
            slot = j % _NBUF

            @pl.when(not_tail)
            def _():
                ob[slot] = res

            @pl.when(is_tail)
            def _():
                ob_t[slot] = res[:tail]

            _o_both(j, lambda d: d.start())
    for j in range(b * nt - _NBUF, b * nt):
        _o_both(j, lambda d: d.wait())


def _logits(idx_row, xbf, c, head_w):
    b, t, e = xbf.shape
    v = head_w.shape[0]
    nv = (v + _VT - 1) // _VT
    out = pl.pallas_call(
        functools.partial(_logits_body, b=b, t=t, v=v, nv=nv),
        grid=(nv,),
        in_specs=[
            pl.BlockSpec((b, 1, t), lambda i: (0, 0, 0)),
            pl.BlockSpec((b, t, e), lambda i: (0, 0, 0)),
            pl.BlockSpec((b, t, t), lambda i: (0, 0, 0)),
            pl.BlockSpec((_VT, e), lambda i: (i, 0)),
        ],
        out_specs=pl.BlockSpec(memory_space=pl.ANY),
        out_shape=jax.ShapeDtypeStruct((v, b, t), _F32),
        scratch_shapes=[
            pltpu.VMEM((_VT, t), _BF16),
            pltpu.VMEM((_NBUF, _VT, _TT), _F32),
            pltpu.VMEM((_NBUF, v - (nv - 1) * _VT, _TT), _F32),
            pltpu.SemaphoreType.DMA((_NBUF,)),
        ],
        compiler_params=pltpu.CompilerParams(
            dimension_semantics=("arbitrary",),
            vmem_limit_bytes=58 * 1024 * 1024,
        ),
        name="logits_copy",
    )(idx_row, xbf, c, head_w)
    return jnp.transpose(out, (1, 2, 0))


def kernel(idx, emb_w, ln_g, ln_b, head_w, head_q_w, head_k_w):
    b, t = idx.shape
    e = emb_w.shape[1]
    idx = idx.astype(jnp.int32)
    x32f, xbff = _gather_ln(
        idx.reshape(-1), emb_w, ln_g.reshape(1, e), ln_b.reshape(1, e)
    )
    x = x32f.reshape(b, t, e)
    xbf = xbff.reshape(b, t, e)
    c = _qkc(xbf, head_q_w.astype(_BF16), head_k_w.astype(_BF16))
    logits = _logits(idx.reshape(b, 1, t), xbf, c, head_w)
    return logits, x


# speedup vs baseline: 1.0027x; 1.0027x over previous
import functools

import jax
import jax.numpy as jnp
from jax.experimental import pallas as pl
from jax.experimental.pallas import tpu as pltpu

LN_EPS = 1e-5

_F32 = jnp.float32
_BF16 = jnp.bfloat16

_G = 512
_TT = 256
_VT = 1024
_NBUF = 2


def _gather_ln_body(idx_ref, emb_ref, g_ref, b_ref, x32_ref, xbf_ref,
                    xg, sem, *, g: int):
    t0 = pl.program_id(0) * g
    for r in range(g):
        pltpu.make_async_copy(
            emb_ref.at[idx_ref[t0 + r]], xg.at[r], sem
        ).start()
    for r in range(g):
        pltpu.make_async_copy(
            emb_ref.at[idx_ref[t0 + r]], xg.at[r], sem
        ).wait()
    rows = xg[...]
    mu = jnp.mean(rows, axis=1, keepdims=True)
    d = rows - mu
    var = jnp.mean(d * d, axis=1, keepdims=True)
    y = d * jax.lax.rsqrt(var + LN_EPS) * g_ref[...] + b_ref[...]
    x32_ref[...] = y
    xbf_ref[...] = y.astype(_BF16)


def _gather_ln(idx_flat, emb_w, ln_g2, ln_b2):
    n, e = idx_flat.shape[0], emb_w.shape[1]
    grid = (n // _G,)
    in_specs = [
        pl.BlockSpec(memory_space=pl.ANY),
        pl.BlockSpec((1, e), lambda i, idx_ref: (0, 0)),
        pl.BlockSpec((1, e), lambda i, idx_ref: (0, 0)),
    ]
    out_specs = [
        pl.BlockSpec((_G, e), lambda i, idx_ref: (i, 0)),
        pl.BlockSpec((_G, e), lambda i, idx_ref: (i, 0)),
    ]
    return pl.pallas_call(
        functools.partial(_gather_ln_body, g=_G),
        grid_spec=pltpu.PrefetchScalarGridSpec(
            num_scalar_prefetch=1,
            grid=grid,
            in_specs=in_specs,
            out_specs=out_specs,
            scratch_shapes=[
                pltpu.VMEM((_G, e), _F32),
                pltpu.SemaphoreType.DMA,
            ],
        ),
        out_shape=[
            jax.ShapeDtypeStruct((n, e), _F32),
            jax.ShapeDtypeStruct((n, e), _BF16),
        ],
        compiler_params=pltpu.CompilerParams(
            dimension_semantics=("arbitrary",),
        ),
        name="gather_ln",
    )(idx_flat, emb_w, ln_g2, ln_b2)


_CDIMS_11 = (((1,), (1,)), ((), ()))
_CDIMS_10 = (((1,), (0,)), ((), ()))


def _qkc_body(x_ref, wq_ref, wk_ref, c_ref, k_s, *, t: int, qk: int):
    k_s[...] = jax.lax.dot_general(
        x_ref[0], wk_ref[...], _CDIMS_11, preferred_element_type=_F32
    ).astype(_BF16)
    inv_qk = _F32(1.0 / qk)
    for i in range(t // _TT):
        sl = slice(i * _TT, (i + 1) * _TT)
        ki = (i + 1) * _TT
        qi = jax.lax.dot_general(
            x_ref[0, sl], wq_ref[...], _CDIMS_11, preferred_element_type=_F32
        ).astype(_BF16)
        ci = jax.lax.dot_general(
            qi, k_s[:ki], _CDIMS_11, preferred_element_type=_F32
        ) * inv_qk
        row = i * _TT + jax.lax.broadcasted_iota(jnp.int32, (_TT, ki), 0)
        col = jax.lax.broadcasted_iota(jnp.int32, (_TT, ki), 1)
        ci = jnp.where(row >= col, ci, _F32(0.0))
        c_ref[0, sl, :ki] = ci.astype(_BF16)
        if ki < t:
            c_ref[0, sl, ki:] = jnp.zeros((_TT, t - ki), _BF16)


def _qkc(xbf, wq, wk):
    b, t, e = xbf.shape
    qk = wq.shape[0]
    return pl.pallas_call(
        functools.partial(_qkc_body, t=t, qk=qk),
        grid=(b,),
        in_specs=[
            pl.BlockSpec((1, t, e), lambda i: (i, 0, 0)),
            pl.BlockSpec((qk, e), lambda i: (0, 0)),
            pl.BlockSpec((qk, e), lambda i: (0, 0)),
        ],
        out_specs=pl.BlockSpec((1, t, t), lambda i: (i, 0, 0)),
        out_shape=jax.ShapeDtypeStruct((b, t, t), _BF16),
        scratch_shapes=[pltpu.VMEM((t, qk), _BF16)],
        compiler_params=pltpu.CompilerParams(
            dimension_semantics=("arbitrary",),
        ),
        name="qkc",
    )(xbf, wq, wk)


def _logits_body(idx_ref, x_ref, c_ref, hw_ref, o_ref,
                 oh_s, ob, ob_t, osem, *, b: int, t: int, v: int, nv: int):
    step = pl.program_id(0)
    v0 = step * _VT
    tail = v - (nv - 1) * _VT
    is_tail = step == nv - 1
    not_tail = step < nv - 1
    hw = hw_ref[...].astype(_BF16)
    col_ids = v0 + jax.lax.broadcasted_iota(jnp.int32, (_VT, t), 0)
    nt = t // _TT

    def _o_both(bi, meth):
        slot = bi % _NBUF

        @pl.when(not_tail)
        def _():
            meth(pltpu.make_async_copy(
                ob.at[slot], o_ref.at[pl.ds(v0, _VT), bi, :], osem.at[slot]))

        @pl.when(is_tail)
        def _():
            meth(pltpu.make_async_copy(
                ob_t.at[slot], o_ref.at[pl.ds(v0, tail), bi, :],
                osem.at[slot]))

    for bi in range(b):
        ids = idx_ref[bi]
        oh_s[...] = jnp.where(ids == col_ids, _F32(1.0), _F32(0.0)).astype(_BF16)
        slot = bi % _NBUF
        if bi >= _NBUF:
            _o_both(bi - _NBUF, lambda d: d.wait())
        for ti in range(nt):
            sl = slice(ti * _TT, (ti + 1) * _TT)
            ki = (ti + 1) * _TT
            head_t = jax.lax.dot_general(
                hw, x_ref[bi, sl], _CDIMS_11, preferred_element_type=_F32
            )
            cpy_t = jax.lax.dot_general(
                oh_s[:, :ki], c_ref[bi, sl, :ki], _CDIMS_11,
                preferred_element_type=_F32,
            )
            res = head_t + cpy_t

            @pl.when(not_tail)
            def _():
                ob[slot, :, sl] = res

            @pl.when(is_tail)
            def _():
                ob_t[slot, :, sl] = res[:tail]

        _o_both(bi, lambda d: d.start())
    for bi in range(max(b - _NBUF, 0), b):
        _o_both(bi, lambda d: d.wait())


def _logits(idx_row, xbf, c, head_w):
    b, t, e = xbf.shape
    v = head_w.shape[0]
    nv = (v + _VT - 1) // _VT
    out = pl.pallas_call(
        functools.partial(_logits_body, b=b, t=t, v=v, nv=nv),
        grid=(nv,),
        in_specs=[
            pl.BlockSpec((b, 1, t), lambda i: (0, 0, 0)),
            pl.BlockSpec((b, t, e), lambda i: (0, 0, 0)),
            pl.BlockSpec((b, t, t), lambda i: (0, 0, 0)),
            pl.BlockSpec((_VT, e), lambda i: (i, 0)),
        ],
        out_specs=pl.BlockSpec(memory_space=pl.ANY),
        out_shape=jax.ShapeDtypeStruct((v, b, t), _F32),
        scratch_shapes=[
            pltpu.VMEM((_VT, t), _BF16),
            pltpu.VMEM((_NBUF, _VT, t), _F32),
            pltpu.VMEM((_NBUF, v - (nv - 1) * _VT, t), _F32),
            pltpu.SemaphoreType.DMA((_NBUF,)),
        ],
        compiler_params=pltpu.CompilerParams(
            dimension_semantics=("arbitrary",),
            vmem_limit_bytes=58 * 1024 * 1024,
        ),
        name="logits_copy",
    )(idx_row, xbf, c, head_w)
    return jnp.transpose(out, (1, 2, 0))


def kernel(idx, emb_w, ln_g, ln_b, head_w, head_q_w, head_k_w):
    b, t = idx.shape
    e = emb_w.shape[1]
    idx = idx.astype(jnp.int32)
    x32f, xbff = _gather_ln(
        idx.reshape(-1), emb_w, ln_g.reshape(1, e), ln_b.reshape(1, e)
    )
    x = x32f.reshape(b, t, e)
    xbf = xbff.reshape(b, t, e)
    c = _qkc(xbf, head_q_w.astype(_BF16), head_k_w.astype(_BF16))
    logits = _logits(idx.reshape(b, 1, t), xbf, c, head_w)
    return logits, x


# speedup vs baseline: 1.6203x; 1.6160x over previous
import functools

import jax
import jax.numpy as jnp
from jax.experimental import pallas as pl
from jax.experimental.pallas import tpu as pltpu

LN_EPS = 1e-5

_F32 = jnp.float32
_BF16 = jnp.bfloat16

_G = 512
_TT = 256
_VT = 1024
_NBUF = 2


def _gather_ln_body(idx_ref, emb_ref, g_ref, b_ref, x32_ref, xbf_ref,
                    xg, sem, *, g: int):
    t0 = pl.program_id(0) * g
    for r in range(g):
        pltpu.make_async_copy(
            emb_ref.at[idx_ref[t0 + r]], xg.at[r], sem
        ).start()
    for r in range(g):
        pltpu.make_async_copy(
            emb_ref.at[idx_ref[t0 + r]], xg.at[r], sem
        ).wait()
    rows = xg[...]
    mu = jnp.mean(rows, axis=1, keepdims=True)
    d = rows - mu
    var = jnp.mean(d * d, axis=1, keepdims=True)
    y = d * jax.lax.rsqrt(var + LN_EPS) * g_ref[...] + b_ref[...]
    x32_ref[...] = y
    xbf_ref[...] = y.astype(_BF16)


def _gather_ln(idx_flat, emb_w, ln_g2, ln_b2):
    n, e = idx_flat.shape[0], emb_w.shape[1]
    grid = (n // _G,)
    in_specs = [
        pl.BlockSpec(memory_space=pl.ANY),
        pl.BlockSpec((1, e), lambda i, idx_ref: (0, 0)),
        pl.BlockSpec((1, e), lambda i, idx_ref: (0, 0)),
    ]
    out_specs = [
        pl.BlockSpec((_G, e), lambda i, idx_ref: (i, 0)),
        pl.BlockSpec((_G, e), lambda i, idx_ref: (i, 0)),
    ]
    return pl.pallas_call(
        functools.partial(_gather_ln_body, g=_G),
        grid_spec=pltpu.PrefetchScalarGridSpec(
            num_scalar_prefetch=1,
            grid=grid,
            in_specs=in_specs,
            out_specs=out_specs,
            scratch_shapes=[
                pltpu.VMEM((_G, e), _F32),
                pltpu.SemaphoreType.DMA,
            ],
        ),
        out_shape=[
            jax.ShapeDtypeStruct((n, e), _F32),
            jax.ShapeDtypeStruct((n, e), _BF16),
        ],
        compiler_params=pltpu.CompilerParams(
            dimension_semantics=("arbitrary",),
        ),
        name="gather_ln",
    )(idx_flat, emb_w, ln_g2, ln_b2)


_CDIMS_11 = (((1,), (1,)), ((), ()))
_CDIMS_10 = (((1,), (0,)), ((), ()))


def _qkc_body(x_ref, wq_ref, wk_ref, c_ref, k_s, *, t: int, qk: int):
    k_s[...] = jax.lax.dot_general(
        x_ref[0], wk_ref[...], _CDIMS_11, preferred_element_type=_F32
    ).astype(_BF16)
    inv_qk = _F32(1.0 / qk)
    for i in range(t // _TT):
        sl = slice(i * _TT, (i + 1) * _TT)
        ki = (i + 1) * _TT
        qi = jax.lax.dot_general(
            x_ref[0, sl], wq_ref[...], _CDIMS_11, preferred_element_type=_F32
        ).astype(_BF16)
        ci = jax.lax.dot_general(
            qi, k_s[:ki], _CDIMS_11, preferred_element_type=_F32
        ) * inv_qk
        row = i * _TT + jax.lax.broadcasted_iota(jnp.int32, (_TT, ki), 0)
        col = jax.lax.broadcasted_iota(jnp.int32, (_TT, ki), 1)
        ci = jnp.where(row >= col, ci, _F32(0.0))
        c_ref[0, sl, :ki] = ci.astype(_BF16)
        if ki < t:
            c_ref[0, sl, ki:] = jnp.zeros((_TT, t - ki), _BF16)


def _qkc(xbf, wq, wk):
    b, t, e = xbf.shape
    qk = wq.shape[0]
    return pl.pallas_call(
        functools.partial(_qkc_body, t=t, qk=qk),
        grid=(b,),
        in_specs=[
            pl.BlockSpec((1, t, e), lambda i: (i, 0, 0)),
            pl.BlockSpec((qk, e), lambda i: (0, 0)),
            pl.BlockSpec((qk, e), lambda i: (0, 0)),
        ],
        out_specs=pl.BlockSpec((1, t, t), lambda i: (i, 0, 0)),
        out_shape=jax.ShapeDtypeStruct((b, t, t), _BF16),
        scratch_shapes=[pltpu.VMEM((t, qk), _BF16)],
        compiler_params=pltpu.CompilerParams(
            dimension_semantics=("arbitrary",),
        ),
        name="qkc",
    )(xbf, wq, wk)


def _compute_tiles(idx_ref, x_ref, c_ref, hw, col_ids, bi, t, emit):
    nt = t // _TT
    for ti in range(nt):
        sl = slice(ti * _TT, (ti + 1) * _TT)
        ki = (ti + 1) * _TT
        head_t = jax.lax.dot_general(
            hw, x_ref[bi, sl], _CDIMS_11, preferred_element_type=_F32
        )
        cpy_t = jax.lax.dot_general(
            col_ids[:, :ki], c_ref[bi, sl, :ki], _CDIMS_11,
            preferred_element_type=_F32,
        )
        emit(sl, head_t + cpy_t)


def _onehot(ids, col_ids):
    return jnp.where(ids == col_ids, _F32(1.0), _F32(0.0)).astype(_BF16)


def _logits_main_body(idx_ref, x_ref, c_ref, hw_ref, o_ref,
                      oh_s, ob, osem, *, b: int, t: int):
    step = pl.program_id(0)
    v0 = step * _VT
    hw = hw_ref[...].astype(_BF16)
    col_ids = v0 + jax.lax.broadcasted_iota(jnp.int32, (_VT, t), 0)

    def _o_dma(bi):
        return pltpu.make_async_copy(
            ob.at[bi % _NBUF], o_ref.at[pl.ds(v0, _VT), bi, :],
            osem.at[bi % _NBUF],
        )

    for bi in range(b):
        oh_s[...] = _onehot(idx_ref[bi], col_ids)
        slot = bi % _NBUF
        if bi >= _NBUF:
            _o_dma(bi - _NBUF).wait()

        def _emit(sl, res, slot=slot):
            ob[slot, :, sl] = res

        _compute_tiles(idx_ref, x_ref, c_ref, hw, oh_s, bi, t, _emit)
        _o_dma(bi).start()
    for bi in range(max(b - _NBUF, 0), b):
        _o_dma(bi).wait()


def _logits_tail_body(idx_ref, x_ref, c_ref, hw_ref, oprev_ref, o_ref, oh_s,
                      *, b: int, t: int, v0: int):
    hw = hw_ref[...].astype(_BF16)
    col_ids = v0 + jax.lax.broadcasted_iota(jnp.int32, (_VT, t), 0)
    for bi in range(b):
        oh_s[...] = _onehot(idx_ref[bi], col_ids)

        def _emit(sl, res, bi=bi):
            o_ref[:, bi, sl] = res

        _compute_tiles(idx_ref, x_ref, c_ref, hw, oh_s, bi, t, _emit)


def _logits(idx_row, xbf, c, head_w):
    b, t, e = xbf.shape
    v = head_w.shape[0]
    nv = (v + _VT - 1) // _VT
    data_specs = [
        pl.BlockSpec((b, 1, t), lambda i: (0, 0, 0)),
        pl.BlockSpec((b, t, e), lambda i: (0, 0, 0)),
        pl.BlockSpec((b, t, t), lambda i: (0, 0, 0)),
    ]
    out = pl.pallas_call(
        functools.partial(_logits_main_body, b=b, t=t),
        grid=(nv - 1,),
        in_specs=data_specs + [pl.BlockSpec((_VT, e), lambda i: (i, 0))],
        out_specs=pl.BlockSpec(memory_space=pl.ANY),
        out_shape=jax.ShapeDtypeStruct((v, b, t), _F32),
        scratch_shapes=[
            pltpu.VMEM((_VT, t), _BF16),
            pltpu.VMEM((_NBUF, _VT, t), _F32),
            pltpu.SemaphoreType.DMA((_NBUF,)),
        ],
        compiler_params=pltpu.CompilerParams(
            dimension_semantics=("arbitrary",),
            vmem_limit_bytes=58 * 1024 * 1024,
        ),
        name="logits_copy",
    )(idx_row, xbf, c, head_w)
    out = pl.pallas_call(
        functools.partial(_logits_tail_body, b=b, t=t, v0=(nv - 1) * _VT),
        grid=(1,),
        in_specs=data_specs + [
            pl.BlockSpec((_VT, e), lambda i: (nv - 1, 0)),
            pl.BlockSpec(memory_space=pl.ANY),
        ],
        out_specs=pl.BlockSpec((_VT, b, t), lambda i: (nv - 1, 0, 0)),
        out_shape=jax.ShapeDtypeStruct((v, b, t), _F32),
        scratch_shapes=[pltpu.VMEM((_VT, t), _BF16)],
        input_output_aliases={4: 0},
        compiler_params=pltpu.CompilerParams(
            dimension_semantics=("arbitrary",),
            vmem_limit_bytes=58 * 1024 * 1024,
        ),
        name="logits_tail",
    )(idx_row, xbf, c, head_w, out)
    return jnp.transpose(out, (1, 2, 0))


def kernel(idx, emb_w, ln_g, ln_b, head_w, head_q_w, head_k_w):
    b, t = idx.shape
    e = emb_w.shape[1]
    idx = idx.astype(jnp.int32)
    x32f, xbff = _gather_ln(
        idx.reshape(-1), emb_w, ln_g.reshape(1, e), ln_b.reshape(1, e)
    )
    x = x32f.reshape(b, t, e)
    xbf = xbff.reshape(b, t, e)
    c = _qkc(xbf, head_q_w.astype(_BF16), head_k_w.astype(_BF16))
    logits = _logits(idx.reshape(b, 1, t), xbf, c, head_w)
    return logits, x


# speedup vs baseline: 1.6384x; 1.0112x over previous
import functools

import jax
import jax.numpy as jnp
from jax.experimental import pallas as pl
from jax.experimental.pallas import tpu as pltpu

LN_EPS = 1e-5

_F32 = jnp.float32
_BF16 = jnp.bfloat16

_G = 512
_TT = 256
_VT = 1024
_NBUF = 4


def _gather_ln_body(idx_ref, emb_ref, g_ref, b_ref, x32_ref, xbf_ref,
                    xg, sem, *, g: int):
    t0 = pl.program_id(0) * g
    for r in range(g):
        pltpu.make_async_copy(
            emb_ref.at[idx_ref[t0 + r]], xg.at[r], sem
        ).start()
    for r in range(g):
        pltpu.make_async_copy(
            emb_ref.at[idx_ref[t0 + r]], xg.at[r], sem
        ).wait()
    rows = xg[...]
    mu = jnp.mean(rows, axis=1, keepdims=True)
    d = rows - mu
    var = jnp.mean(d * d, axis=1, keepdims=True)
    y = d * jax.lax.rsqrt(var + LN_EPS) * g_ref[...] + b_ref[...]
    x32_ref[...] = y
    xbf_ref[...] = y.astype(_BF16)


def _gather_ln(idx_flat, emb_w, ln_g2, ln_b2):
    n, e = idx_flat.shape[0], emb_w.shape[1]
    grid = (n // _G,)
    in_specs = [
        pl.BlockSpec(memory_space=pl.ANY),
        pl.BlockSpec((1, e), lambda i, idx_ref: (0, 0)),
        pl.BlockSpec((1, e), lambda i, idx_ref: (0, 0)),
    ]
    out_specs = [
        pl.BlockSpec((_G, e), lambda i, idx_ref: (i, 0)),
        pl.BlockSpec((_G, e), lambda i, idx_ref: (i, 0)),
    ]
    return pl.pallas_call(
        functools.partial(_gather_ln_body, g=_G),
        grid_spec=pltpu.PrefetchScalarGridSpec(
            num_scalar_prefetch=1,
            grid=grid,
            in_specs=in_specs,
            out_specs=out_specs,
            scratch_shapes=[
                pltpu.VMEM((_G, e), _F32),
                pltpu.SemaphoreType.DMA,
            ],
        ),
        out_shape=[
            jax.ShapeDtypeStruct((n, e), _F32),
            jax.ShapeDtypeStruct((n, e), _BF16),
        ],
        compiler_params=pltpu.CompilerParams(
            dimension_semantics=("arbitrary",),
        ),
        name="gather_ln",
    )(idx_flat, emb_w, ln_g2, ln_b2)


_CDIMS_11 = (((1,), (1,)), ((), ()))
_CDIMS_10 = (((1,), (0,)), ((), ()))


def _qkc_body(x_ref, wq_ref, wk_ref, c_ref, k_s, *, t: int, qk: int):
    k_s[...] = jax.lax.dot_general(
        x_ref[0], wk_ref[...], _CDIMS_11, preferred_element_type=_F32
    ).astype(_BF16)
    inv_qk = _F32(1.0 / qk)
    for i in range(t // _TT):
        sl = slice(i * _TT, (i + 1) * _TT)
        ki = (i + 1) * _TT
        qi = jax.lax.dot_general(
            x_ref[0, sl], wq_ref[...], _CDIMS_11, preferred_element_type=_F32
        ).astype(_BF16)
        ci = jax.lax.dot_general(
            qi, k_s[:ki], _CDIMS_11, preferred_element_type=_F32
        ) * inv_qk
        row = i * _TT + jax.lax.broadcasted_iota(jnp.int32, (_TT, ki), 0)
        col = jax.lax.broadcasted_iota(jnp.int32, (_TT, ki), 1)
        ci = jnp.where(row >= col, ci, _F32(0.0))
        c_ref[0, sl, :ki] = ci.astype(_BF16)
        if ki < t:
            c_ref[0, sl, ki:] = jnp.zeros((_TT, t - ki), _BF16)


def _qkc(xbf, wq, wk):
    b, t, e = xbf.shape
    qk = wq.shape[0]
    return pl.pallas_call(
        functools.partial(_qkc_body, t=t, qk=qk),
        grid=(b,),
        in_specs=[
            pl.BlockSpec((1, t, e), lambda i: (i, 0, 0)),
            pl.BlockSpec((qk, e), lambda i: (0, 0)),
            pl.BlockSpec((qk, e), lambda i: (0, 0)),
        ],
        out_specs=pl.BlockSpec((1, t, t), lambda i: (i, 0, 0)),
        out_shape=jax.ShapeDtypeStruct((b, t, t), _BF16),
        scratch_shapes=[pltpu.VMEM((t, qk), _BF16)],
        compiler_params=pltpu.CompilerParams(
            dimension_semantics=("arbitrary",),
        ),
        name="qkc",
    )(xbf, wq, wk)


def _compute_tiles(idx_ref, x_ref, c_ref, hw, col_ids, bi, t, emit):
    nt = t // _TT
    for ti in range(nt):
        sl = slice(ti * _TT, (ti + 1) * _TT)
        ki = (ti + 1) * _TT
        head_t = jax.lax.dot_general(
            hw, x_ref[bi, sl], _CDIMS_11, preferred_element_type=_F32
        )
        cpy_t = jax.lax.dot_general(
            col_ids[:, :ki], c_ref[bi, sl, :ki], _CDIMS_11,
            preferred_element_type=_F32,
        )
        emit(sl, head_t + cpy_t)


def _onehot(ids, col_ids):
    return jnp.where(ids == col_ids, _F32(1.0), _F32(0.0)).astype(_BF16)


def _logits_main_body(idx_ref, x_ref, c_ref, hw_ref, o_ref,
                      oh_s, ob, osem, *, b: int, t: int):
    step = pl.program_id(0)
    v0 = step * _VT
    hw = hw_ref[...].astype(_BF16)
    col_ids = v0 + jax.lax.broadcasted_iota(jnp.int32, (_VT, t), 0)

    def _o_dma(bi):
        return pltpu.make_async_copy(
            ob.at[bi % _NBUF], o_ref.at[pl.ds(v0, _VT), bi, :],
            osem.at[bi % _NBUF],
        )

    nsteps = pl.num_programs(0)
    for bi in range(b):
        oh_s[...] = _onehot(idx_ref[bi], col_ids)

        @pl.when(step > 0)
        def _():
            _o_dma(bi).wait()

        def _emit(sl, res, bi=bi):
            ob[bi % _NBUF, :, sl] = res

        _compute_tiles(idx_ref, x_ref, c_ref, hw, oh_s, bi, t, _emit)
        _o_dma(bi).start()

    @pl.when(step == nsteps - 1)
    def _():
        for bi in range(b):
            _o_dma(bi).wait()


def _logits_tail_body(idx_ref, x_ref, c_ref, hw_ref, oprev_ref, o_ref, oh_s,
                      *, b: int, t: int, v0: int):
    hw = hw_ref[...].astype(_BF16)
    col_ids = v0 + jax.lax.broadcasted_iota(jnp.int32, (_VT, t), 0)
    for bi in range(b):
        oh_s[...] = _onehot(idx_ref[bi], col_ids)

        def _emit(sl, res, bi=bi):
            o_ref[:, bi, sl] = res

        _compute_tiles(idx_ref, x_ref, c_ref, hw, oh_s, bi, t, _emit)


def _logits(idx_row, xbf, c, head_w):
    b, t, e = xbf.shape
    v = head_w.shape[0]
    nv = (v + _VT - 1) // _VT
    data_specs = [
        pl.BlockSpec((b, 1, t), lambda i: (0, 0, 0)),
        pl.BlockSpec((b, t, e), lambda i: (0, 0, 0)),
        pl.BlockSpec((b, t, t), lambda i: (0, 0, 0)),
    ]
    out = pl.pallas_call(
        functools.partial(_logits_main_body, b=b, t=t),
        grid=(nv - 1,),
        in_specs=data_specs + [pl.BlockSpec((_VT, e), lambda i: (i, 0))],
        out_specs=pl.BlockSpec(memory_space=pl.ANY),
        out_shape=jax.ShapeDtypeStruct((v, b, t), _F32),
        scratch_shapes=[
            pltpu.VMEM((_VT, t), _BF16),
            pltpu.VMEM((_NBUF, _VT, t), _F32),
            pltpu.SemaphoreType.DMA((_NBUF,)),
        ],
        compiler_params=pltpu.CompilerParams(
            dimension_semantics=("arbitrary",),
            vmem_limit_bytes=58 * 1024 * 1024,
        ),
        name="logits_copy",
    )(idx_row, xbf, c, head_w)
    out = pl.pallas_call(
        functools.partial(_logits_tail_body, b=b, t=t, v0=(nv - 1) * _VT),
        grid=(1,),
        in_specs=data_specs + [
            pl.BlockSpec((_VT, e), lambda i: (nv - 1, 0)),
            pl.BlockSpec(memory_space=pl.ANY),
        ],
        out_specs=pl.BlockSpec((_VT, b, t), lambda i: (nv - 1, 0, 0)),
        out_shape=jax.ShapeDtypeStruct((v, b, t), _F32),
        scratch_shapes=[pltpu.VMEM((_VT, t), _BF16)],
        input_output_aliases={4: 0},
        compiler_params=pltpu.CompilerParams(
            dimension_semantics=("arbitrary",),
            vmem_limit_bytes=58 * 1024 * 1024,
        ),
        name="logits_tail",
    )(idx_row, xbf, c, head_w, out)
    return jnp.transpose(out, (1, 2, 0))


def kernel(idx, emb_w, ln_g, ln_b, head_w, head_q_w, head_k_w):
    b, t = idx.shape
    e = emb_w.shape[1]
    idx = idx.astype(jnp.int32)
    x32f, xbff = _gather_ln(
        idx.reshape(-1), emb_w, ln_g.reshape(1, e), ln_b.reshape(1, e)
    )
    x = x32f.reshape(b, t, e)
    xbf = xbff.reshape(b, t, e)
    c = _qkc(xbf, head_q_w.astype(_BF16), head_k_w.astype(_BF16))
    logits = _logits(idx.reshape(b, 1, t), xbf, c, head_w)
    return logits, x
